# baseline (speedup 1.0000x reference)
import functools

import jax
import jax.numpy as jnp
from jax.experimental import pallas as pl
from jax.experimental.pallas import tpu as pltpu

D = 768
E = 8
K = 2
DFF = 3072

BM = 512
BF = 768
NJ = DFF // BF
NT = 16
RMAX = NT * BM


def _gmm_body(expert_ref, active_ref, xs_ref, wg_ref, wu_ref, wd_ref,
              out_ref, yacc_ref):
    i = pl.program_id(0)
    j = pl.program_id(1)

    @pl.when(active_ref[i] > 0)
    def _():
        x = xs_ref[...]
        g = jax.lax.dot_general(x, wg_ref[0], (((1,), (1,)), ((), ())),
                                preferred_element_type=jnp.float32)
        u = jax.lax.dot_general(x, wu_ref[0], (((1,), (1,)), ((), ())),
                                preferred_element_type=jnp.float32)
        h = (jnp.maximum(g, 0.0) * u).astype(jnp.bfloat16)
        y = jax.lax.dot_general(h, wd_ref[0], (((1,), (1,)), ((), ())),
                                preferred_element_type=jnp.float32)

        @pl.when(j == 0)
        def _():
            yacc_ref[...] = y

        @pl.when(j > 0)
        def _():
            yacc_ref[...] += y

        @pl.when(j == NJ - 1)
        def _():
            out_ref[...] = yacc_ref[...]


@functools.partial(jax.jit, static_argnames=())
def _grouped_glu(texp, active, xs, Wg, Wu, Wd):
    grid_spec = pltpu.PrefetchScalarGridSpec(
        num_scalar_prefetch=2,
        grid=(NT, NJ),
        in_specs=[
            pl.BlockSpec((BM, D), lambda i, j, er, ar: (i, 0)),
            pl.BlockSpec((1, BF, D), lambda i, j, er, ar: (er[i], j, 0)),
            pl.BlockSpec((1, BF, D), lambda i, j, er, ar: (er[i], j, 0)),
            pl.BlockSpec((1, D, BF), lambda i, j, er, ar: (er[i], 0, j)),
        ],
        out_specs=pl.BlockSpec((BM, D), lambda i, j, er, ar: (i, 0)),
        scratch_shapes=[pltpu.VMEM((BM, D), jnp.float32)],
    )
    return pl.pallas_call(
        _gmm_body,
        grid_spec=grid_spec,
        out_shape=jax.ShapeDtypeStruct((RMAX, D), jnp.float32),
        compiler_params=pltpu.CompilerParams(
            dimension_semantics=("arbitrary", "arbitrary"),
        ),
    )(texp, active, xs, Wg, Wu, Wd)


def kernel(x, Wr, Wg, Wu, Wd):
    Bb, Ll, Dd = x.shape
    T = Bb * Ll
    TK = T * K
    xf = x.reshape(T, Dd)

    logits = xf @ Wr.T
    probs = jax.nn.softmax(logits, axis=-1)
    e1 = jnp.argmax(probs, axis=-1).astype(jnp.int32)
    p1 = jnp.max(probs, axis=-1)
    lane = jnp.arange(E, dtype=jnp.int32)
    probs2 = jnp.where(lane[None, :] == e1[:, None], -jnp.inf, probs)
    e2 = jnp.argmax(probs2, axis=-1).astype(jnp.int32)
    p2 = jnp.max(probs2, axis=-1)
    s = p1 + p2
    k_w = jnp.stack([p1 / s, p2 / s], axis=1)
    e_flat = jnp.concatenate([e1, e2])

    onehot = (e_flat[:, None] == lane[None, :]).astype(jnp.int32)
    counts = jnp.sum(onehot, axis=0)
    padded = ((counts + BM - 1) // BM) * BM
    gstart = jnp.concatenate([jnp.zeros(1, jnp.int32),
                              jnp.cumsum(padded)[:-1].astype(jnp.int32)])
    cstart = jnp.concatenate([jnp.zeros(1, jnp.int32),
                              jnp.cumsum(counts)[:-1].astype(jnp.int32)])
    order = jnp.argsort(e_flat, stable=True).astype(jnp.int32)
    e_sorted = e_flat[order]
    rank = jnp.arange(TK, dtype=jnp.int32) - cstart[e_sorted]
    dest_sorted = gstart[e_sorted] + rank
    row_src = jnp.zeros(RMAX, jnp.int32).at[dest_sorted].set(order % T)
    inv = jnp.zeros(TK, jnp.int32).at[order].set(dest_sorted)

    ptotal = jnp.sum(padded)
    n_active = ptotal // BM
    tiles = jnp.arange(NT, dtype=jnp.int32)
    gend = (gstart + padded).astype(jnp.int32)
    texp = jnp.sum((tiles[:, None] * BM >= gend[None, :]).astype(jnp.int32),
                   axis=1)
    texp = jnp.minimum(texp, E - 1).astype(jnp.int32)
    active = (tiles < n_active).astype(jnp.int32)

    xs = xf.astype(jnp.bfloat16)[row_src]
    y = _grouped_glu(texp, active, xs,
                     Wg.astype(jnp.bfloat16), Wu.astype(jnp.bfloat16),
                     Wd.astype(jnp.bfloat16))

    yk = y[inv.reshape(K, T)]
    out = jnp.sum(yk * k_w.T[:, :, None], axis=0)
    return out.reshape(Bb, Ll, Dd)


# speedup vs baseline: 1.1657x; 1.1657x over previous
import functools

import jax
import jax.numpy as jnp
from jax.experimental import pallas as pl
from jax.experimental.pallas import tpu as pltpu

D = 768
E = 8
K = 2
DFF = 3072

BM = 1024
BF = 768
NJ = DFF // BF
NT = 12
RMAX = NT * BM


def _gmm_body(expert_ref, active_ref, xs_ref, wg_ref, wu_ref, wd_ref,
              out_ref, yacc_ref):
    i = pl.program_id(0)
    j = pl.program_id(1)

    @pl.when(active_ref[i] > 0)
    def _():
        x = xs_ref[...]
        g = jax.lax.dot_general(x, wg_ref[0].astype(jnp.bfloat16),
                                (((1,), (1,)), ((), ())),
                                preferred_element_type=jnp.float32)
        u = jax.lax.dot_general(x, wu_ref[0].astype(jnp.bfloat16),
                                (((1,), (1,)), ((), ())),
                                preferred_element_type=jnp.float32)
        h = (jnp.maximum(g, 0.0) * u).astype(jnp.bfloat16)
        y = jax.lax.dot_general(h, wd_ref[0].astype(jnp.bfloat16),
                                (((1,), (1,)), ((), ())),
                                preferred_element_type=jnp.float32)

        @pl.when(j == 0)
        def _():
            yacc_ref[...] = y

        @pl.when(j > 0)
        def _():
            yacc_ref[...] += y

        @pl.when(j == NJ - 1)
        def _():
            out_ref[...] = yacc_ref[...]


@functools.partial(jax.jit, static_argnames=())
def _grouped_glu(texp, active, xs, Wg, Wu, Wd):
    grid_spec = pltpu.PrefetchScalarGridSpec(
        num_scalar_prefetch=2,
        grid=(NT, NJ),
        in_specs=[
            pl.BlockSpec((BM, D), lambda i, j, er, ar: (i, 0)),
            pl.BlockSpec((1, BF, D), lambda i, j, er, ar: (er[i], j, 0)),
            pl.BlockSpec((1, BF, D), lambda i, j, er, ar: (er[i], j, 0)),
            pl.BlockSpec((1, D, BF), lambda i, j, er, ar: (er[i], 0, j)),
        ],
        out_specs=pl.BlockSpec((BM, D), lambda i, j, er, ar: (i, 0)),
        scratch_shapes=[pltpu.VMEM((BM, D), jnp.float32)],
    )
    return pl.pallas_call(
        _gmm_body,
        grid_spec=grid_spec,
        out_shape=jax.ShapeDtypeStruct((RMAX, D), jnp.float32),
        compiler_params=pltpu.CompilerParams(
            dimension_semantics=("arbitrary", "arbitrary"),
        ),
    )(texp, active, xs, Wg, Wu, Wd)


def kernel(x, Wr, Wg, Wu, Wd):
    Bb, Ll, Dd = x.shape
    T = Bb * Ll
    TK = T * K
    xf = x.reshape(T, Dd)

    logits = xf @ Wr.T
    probs = jax.nn.softmax(logits, axis=-1)
    e1 = jnp.argmax(probs, axis=-1).astype(jnp.int32)
    p1 = jnp.max(probs, axis=-1)
    lane = jnp.arange(E, dtype=jnp.int32)
    probs2 = jnp.where(lane[None, :] == e1[:, None], -jnp.inf, probs)
    e2 = jnp.argmax(probs2, axis=-1).astype(jnp.int32)
    p2 = jnp.max(probs2, axis=-1)
    s = p1 + p2
    k_w = jnp.stack([p1 / s, p2 / s], axis=1)
    e_flat = jnp.concatenate([e1, e2])

    onehot = (e_flat[:, None] == lane[None, :]).astype(jnp.int32)
    csum = jnp.cumsum(onehot, axis=0)
    counts = csum[-1]
    padded = ((counts + BM - 1) // BM) * BM
    gstart = jnp.concatenate([jnp.zeros(1, jnp.int32),
                              jnp.cumsum(padded)[:-1].astype(jnp.int32)])
    rank = jnp.take_along_axis(csum, e_flat[:, None], axis=1)[:, 0] - 1
    dest = gstart[e_flat] + rank
    tok = jnp.arange(TK, dtype=jnp.int32) % T
    row_src = jnp.zeros(RMAX, jnp.int32).at[dest].set(tok)
    inv = dest

    ptotal = jnp.sum(padded)
    n_active = ptotal // BM
    tiles = jnp.arange(NT, dtype=jnp.int32)
    gend = (gstart + padded).astype(jnp.int32)
    texp = jnp.sum((tiles[:, None] * BM >= gend[None, :]).astype(jnp.int32),
                   axis=1)
    texp = jnp.minimum(texp, E - 1).astype(jnp.int32)
    active = (tiles < n_active).astype(jnp.int32)

    xs = xf.astype(jnp.bfloat16)[row_src]
    y = _grouped_glu(texp, active, xs, Wg, Wu, Wd)

    yk = y[inv.reshape(K, T)]
    out = jnp.sum(yk * k_w.T[:, :, None], axis=0)
    return out.reshape(Bb, Ll, Dd)


# speedup vs baseline: 1.1778x; 1.0104x over previous
import functools

import jax
import jax.numpy as jnp
from jax.experimental import pallas as pl
from jax.experimental.pallas import tpu as pltpu

D = 768
E = 8
K = 2
DFF = 3072

BM = 1024
BF = 768
NJ = DFF // BF
NT = 12
RMAX = NT * BM


def _gmm_body(expert_ref, clampi_ref, xs_ref, wg_ref, wu_ref, wd_ref,
              out_ref, yacc_ref):
    i = pl.program_id(0)
    j = pl.program_id(1)

    @pl.when(clampi_ref[i] == i)
    def _():
        x = xs_ref[...]
        g = jax.lax.dot_general(x, wg_ref[0], (((1,), (1,)), ((), ())),
                                preferred_element_type=jnp.float32)
        u = jax.lax.dot_general(x, wu_ref[0], (((1,), (1,)), ((), ())),
                                preferred_element_type=jnp.float32)
        h = jnp.maximum(g, 0.0) * u
        y = jax.lax.dot_general(h, wd_ref[0], (((1,), (1,)), ((), ())),
                                preferred_element_type=jnp.float32)

        @pl.when(j == 0)
        def _():
            yacc_ref[...] = y

        @pl.when(j > 0)
        def _():
            yacc_ref[...] += y

        @pl.when(j == NJ - 1)
        def _():
            out_ref[...] = yacc_ref[...]


@functools.partial(jax.jit, static_argnames=())
def _grouped_glu(texp, active, xs, Wg, Wu, Wd):
    grid_spec = pltpu.PrefetchScalarGridSpec(
        num_scalar_prefetch=2,
        grid=(NT, NJ),
        in_specs=[
            pl.BlockSpec((BM, D), lambda i, j, er, ci: (ci[i], 0)),
            pl.BlockSpec((1, BF, D), lambda i, j, er, ci: (er[i], j, 0)),
            pl.BlockSpec((1, BF, D), lambda i, j, er, ci: (er[i], j, 0)),
            pl.BlockSpec((1, D, BF), lambda i, j, er, ci: (er[i], 0, j)),
        ],
        out_specs=pl.BlockSpec((BM, D), lambda i, j, er, ci: (ci[i], 0)),
        scratch_shapes=[pltpu.VMEM((BM, D), jnp.float32)],
    )
    return pl.pallas_call(
        _gmm_body,
        grid_spec=grid_spec,
        out_shape=jax.ShapeDtypeStruct((RMAX, D), jnp.float32),
        compiler_params=pltpu.CompilerParams(
            dimension_semantics=("arbitrary", "arbitrary"),
        ),
    )(texp, active, xs, Wg, Wu, Wd)


def kernel(x, Wr, Wg, Wu, Wd):
    Bb, Ll, Dd = x.shape
    T = Bb * Ll
    TK = T * K
    xf = x.reshape(T, Dd)

    logits = xf @ Wr.T
    probs = jax.nn.softmax(logits, axis=-1)
    e1 = jnp.argmax(probs, axis=-1).astype(jnp.int32)
    p1 = jnp.max(probs, axis=-1)
    lane = jnp.arange(E, dtype=jnp.int32)
    probs2 = jnp.where(lane[None, :] == e1[:, None], -jnp.inf, probs)
    e2 = jnp.argmax(probs2, axis=-1).astype(jnp.int32)
    p2 = jnp.max(probs2, axis=-1)
    s = p1 + p2
    k_w = jnp.stack([p1 / s, p2 / s], axis=1)
    e_flat = jnp.concatenate([e1, e2])

    onehot = (e_flat[:, None] == lane[None, :]).astype(jnp.int32)
    csum = jnp.cumsum(onehot, axis=0)
    counts = csum[-1]
    padded = ((counts + BM - 1) // BM) * BM
    gstart = jnp.concatenate([jnp.zeros(1, jnp.int32),
                              jnp.cumsum(padded)[:-1].astype(jnp.int32)])
    rank = jnp.take_along_axis(csum, e_flat[:, None], axis=1)[:, 0] - 1
    dest = gstart[e_flat] + rank
    tok = jnp.arange(TK, dtype=jnp.int32) % T
    row_src = jnp.zeros(RMAX, jnp.int32).at[dest].set(tok)
    inv = dest

    ptotal = jnp.sum(padded)
    n_active = ptotal // BM
    tiles = jnp.arange(NT, dtype=jnp.int32)
    gend = (gstart + padded).astype(jnp.int32)
    texp = jnp.sum((tiles[:, None] * BM >= gend[None, :]).astype(jnp.int32),
                   axis=1)
    texp = jnp.minimum(texp, E - 1).astype(jnp.int32)
    clampi = jnp.minimum(tiles, n_active - 1).astype(jnp.int32)
    texp = texp[clampi]

    xs = xf[row_src]
    y = _grouped_glu(texp, clampi, xs, Wg, Wu, Wd)

    yk = y[inv.reshape(K, T)]
    out = jnp.sum(yk * k_w.T[:, :, None], axis=0)
    return out.reshape(Bb, Ll, Dd)


# speedup vs baseline: 1.2249x; 1.0400x over previous
import functools

import jax
import jax.numpy as jnp
from jax.experimental import pallas as pl
from jax.experimental.pallas import tpu as pltpu

D = 768
E = 8
K = 2
DFF = 3072

BM = 1024
BF = 1536
NJ = DFF // BF
NT = 12
RMAX = NT * BM


def _gmm_body(expert_ref, clampi_ref, xs_ref, wg_ref, wu_ref, wd_ref,
              out_ref, yacc_ref):
    i = pl.program_id(0)
    j = pl.program_id(1)

    @pl.when(clampi_ref[i] == i)
    def _():
        x = xs_ref[...]
        g = jax.lax.dot_general(x, wg_ref[0], (((1,), (1,)), ((), ())),
                                preferred_element_type=jnp.float32)
        u = jax.lax.dot_general(x, wu_ref[0], (((1,), (1,)), ((), ())),
                                preferred_element_type=jnp.float32)
        h = jnp.maximum(g, 0.0) * u
        y = jax.lax.dot_general(h, wd_ref[0], (((1,), (1,)), ((), ())),
                                preferred_element_type=jnp.float32)

        @pl.when(j == 0)
        def _():
            yacc_ref[...] = y

        @pl.when(j > 0)
        def _():
            yacc_ref[...] += y

        @pl.when(j == NJ - 1)
        def _():
            out_ref[...] = yacc_ref[...]


@functools.partial(jax.jit, static_argnames=())
def _grouped_glu(texp, active, xs, Wg, Wu, Wd):
    grid_spec = pltpu.PrefetchScalarGridSpec(
        num_scalar_prefetch=2,
        grid=(NT, NJ),
        in_specs=[
            pl.BlockSpec((BM, D), lambda i, j, er, ci: (ci[i], 0)),
            pl.BlockSpec((1, BF, D), lambda i, j, er, ci: (er[i], j, 0)),
            pl.BlockSpec((1, BF, D), lambda i, j, er, ci: (er[i], j, 0)),
            pl.BlockSpec((1, D, BF), lambda i, j, er, ci: (er[i], 0, j)),
        ],
        out_specs=pl.BlockSpec((BM, D), lambda i, j, er, ci: (ci[i], 0)),
        scratch_shapes=[pltpu.VMEM((BM, D), jnp.float32)],
    )
    return pl.pallas_call(
        _gmm_body,
        grid_spec=grid_spec,
        out_shape=jax.ShapeDtypeStruct((RMAX, D), jnp.float32),
        compiler_params=pltpu.CompilerParams(
            dimension_semantics=("arbitrary", "arbitrary"),
        ),
    )(texp, active, xs, Wg, Wu, Wd)


def kernel(x, Wr, Wg, Wu, Wd):
    Bb, Ll, Dd = x.shape
    T = Bb * Ll
    TK = T * K
    xf = x.reshape(T, Dd)

    logits = xf @ Wr.T
    probs = jax.nn.softmax(logits, axis=-1)
    e1 = jnp.argmax(probs, axis=-1).astype(jnp.int32)
    p1 = jnp.max(probs, axis=-1)
    lane = jnp.arange(E, dtype=jnp.int32)
    probs2 = jnp.where(lane[None, :] == e1[:, None], -jnp.inf, probs)
    e2 = jnp.argmax(probs2, axis=-1).astype(jnp.int32)
    p2 = jnp.max(probs2, axis=-1)
    s = p1 + p2
    k_w = jnp.stack([p1 / s, p2 / s], axis=1)
    e_flat = jnp.concatenate([e1, e2])

    onehot = (e_flat[:, None] == lane[None, :]).astype(jnp.int32)
    csum = jnp.cumsum(onehot, axis=0)
    counts = csum[-1]
    padded = ((counts + BM - 1) // BM) * BM
    gstart = jnp.concatenate([jnp.zeros(1, jnp.int32),
                              jnp.cumsum(padded)[:-1].astype(jnp.int32)])
    rank = jnp.take_along_axis(csum, e_flat[:, None], axis=1)[:, 0] - 1
    dest = gstart[e_flat] + rank
    tok = jnp.arange(TK, dtype=jnp.int32) % T
    row_src = jnp.zeros(RMAX, jnp.int32).at[dest].set(tok)
    inv = dest

    ptotal = jnp.sum(padded)
    n_active = ptotal // BM
    tiles = jnp.arange(NT, dtype=jnp.int32)
    gend = (gstart + padded).astype(jnp.int32)
    texp = jnp.sum((tiles[:, None] * BM >= gend[None, :]).astype(jnp.int32),
                   axis=1)
    texp = jnp.minimum(texp, E - 1).astype(jnp.int32)
    clampi = jnp.minimum(tiles, n_active - 1).astype(jnp.int32)
    texp = texp[clampi]

    xs = xf[row_src]
    y = _grouped_glu(texp, clampi, xs, Wg, Wu, Wd)

    yk = y[inv.reshape(K, T)]
    out = jnp.sum(yk * k_w.T[:, :, None], axis=0)
    return out.reshape(Bb, Ll, Dd)
